# baseline (speedup 1.0000x reference)
import functools
import math

import jax
import jax.numpy as jnp
from jax import lax
from jax.experimental import pallas as pl
from jax.experimental.pallas import tpu as pltpu
from jax.experimental.pallas import tpu_sc as plsc

VOCAB = 100000
D_MODEL = 768
BATCH = 4
SEQ = 8192
TOK = BATCH * SEQ
NC, NS, LANES = 2, 16, 16
NW = NC * NS
TPW = TOK // NW
K = 32
N_CHUNKS = TPW // K
SCALE = math.sqrt(D_MODEL)


def _pe_table():
    pos = jnp.arange(SEQ, dtype=jnp.float32)[:, None]
    i = jnp.arange(D_MODEL // 2, dtype=jnp.float32)[None, :]
    angle = pos / jnp.power(10000.0, (2.0 * i) / D_MODEL)
    pe = jnp.zeros((SEQ, D_MODEL), dtype=jnp.float32)
    pe = pe.at[:, 0::2].set(jnp.sin(angle))
    pe = pe.at[:, 1::2].set(jnp.cos(angle))
    return pe


@functools.partial(
    pl.kernel,
    mesh=plsc.VectorSubcoreMesh(core_axis_name="c", subcore_axis_name="s"),
    out_type=jax.ShapeDtypeStruct((TOK, D_MODEL), jnp.float32),
    scratch_types=[
        pltpu.VMEM((K,), jnp.int32),
        pltpu.VMEM((K, D_MODEL), jnp.float32),
        pltpu.VMEM((K, D_MODEL), jnp.float32),
        pltpu.SemaphoreType.DMA,
    ],
)
def _embed_sc(table_hbm, idx_hbm, pe_hbm, out_hbm, idx_v, rows_v, pe_v, sem):
    wid = lax.axis_index("s") * NC + lax.axis_index("c")
    base = wid * TPW
    pos0 = lax.rem(base, SEQ)

    def chunk_body(j, carry):
        off = pl.multiple_of(base + j * K, K)
        poff = pl.multiple_of(pos0 + j * K, K)
        pltpu.sync_copy(idx_hbm.at[pl.ds(off, K)], idx_v)
        gather = pltpu.async_copy(table_hbm.at[idx_v], rows_v, sem)
        pltpu.sync_copy(pe_hbm.at[pl.ds(poff, K)], pe_v)
        gather.wait()

        def row_body(r, rcarry):
            for l in range(D_MODEL // LANES):
                sl = pl.ds(l * LANES, LANES)
                rows_v[r, sl] = rows_v[r, sl] * SCALE + pe_v[r, sl]
            return rcarry

        lax.fori_loop(0, K, row_body, 0)
        pltpu.sync_copy(rows_v, out_hbm.at[pl.ds(off, K)])
        return carry

    lax.fori_loop(0, N_CHUNKS, chunk_body, 0)


def kernel(x, table):
    idx = x.reshape(TOK).astype(jnp.int32)
    out = _embed_sc(table, idx, _pe_table())
    return out.reshape(BATCH, SEQ, D_MODEL)


# speedup vs baseline: 1.2600x; 1.2600x over previous
import functools
import math

import jax
import jax.numpy as jnp
from jax import lax
from jax.experimental import pallas as pl
from jax.experimental.pallas import tpu as pltpu
from jax.experimental.pallas import tpu_sc as plsc

VOCAB = 100000
D_MODEL = 768
BATCH = 4
SEQ = 8192
TOK = BATCH * SEQ
NC, NS, LANES = 2, 16, 16
NW = NC * NS
PPW = SEQ // NW
KP = 32
NP = PPW // KP
TASKS = 2 * BATCH
SCALE = math.sqrt(D_MODEL)


def _pe_table():
    pos = jnp.arange(SEQ, dtype=jnp.float32)[:, None]
    i = jnp.arange(D_MODEL // 2, dtype=jnp.float32)[None, :]
    angle = pos / jnp.power(10000.0, (2.0 * i) / D_MODEL)
    pe = jnp.zeros((SEQ, D_MODEL), dtype=jnp.float32)
    pe = pe.at[:, 0::2].set(jnp.sin(angle))
    pe = pe.at[:, 1::2].set(jnp.cos(angle))
    return pe


@functools.partial(
    pl.kernel,
    mesh=plsc.VectorSubcoreMesh(core_axis_name="c", subcore_axis_name="s"),
    out_type=jax.ShapeDtypeStruct((TOK, D_MODEL), jnp.float32),
    scratch_types=[
        pltpu.VMEM((KP,), jnp.int32),
        pltpu.VMEM((KP,), jnp.int32),
        pltpu.VMEM((KP, D_MODEL), jnp.float32),
        pltpu.VMEM((KP, D_MODEL), jnp.float32),
        pltpu.VMEM((KP, D_MODEL), jnp.float32),
        pltpu.VMEM((KP, D_MODEL), jnp.float32),
        pltpu.SemaphoreType.DMA,
        pltpu.SemaphoreType.DMA,
        pltpu.SemaphoreType.DMA,
        pltpu.SemaphoreType.DMA,
        pltpu.SemaphoreType.DMA,
        pltpu.SemaphoreType.DMA,
    ],
)
def _embed_sc(table_hbm, idx_hbm, pe_hbm, out_hbm,
              idx0, idx1, rows0, rows1, pe0, pe1,
              sg0, sg1, ss0, ss1, spe0, spe1):
    idxb, rowsb, peb = [idx0, idx1], [rows0, rows1], [pe0, pe1]
    sgb, ssb, speb = [sg0, sg1], [ss0, ss1], [spe0, spe1]

    wid = lax.axis_index("s") * NC + lax.axis_index("c")
    pbase = wid * PPW

    def outer(p2, carry):
        p = 2 * p2
        pos_off = [pl.multiple_of(pbase + (p + pp) * KP, KP) for pp in range(2)]
        pe_cp = [
            pltpu.async_copy(pe_hbm.at[pl.ds(pos_off[pp], KP)], peb[pp], speb[pp])
            for pp in range(2)
        ]

        def start_gather(t):
            pp, b = t // BATCH, t % BATCH
            off = pl.multiple_of(b * SEQ + pos_off[pp], KP)
            pltpu.sync_copy(idx_hbm.at[pl.ds(off, KP)], idxb[t % 2])
            return pltpu.async_copy(table_hbm.at[idxb[t % 2]], rowsb[t % 2],
                                    sgb[t % 2]), off

        gather = [None] * TASKS
        offs = [None] * TASKS
        store = [None] * TASKS
        gather[0], offs[0] = start_gather(0)
        for t in range(TASKS):
            pp = t // BATCH
            if t % BATCH == 0:
                pe_cp[pp].wait()
            if t + 1 < TASKS:
                if t >= 1:
                    store[t - 1].wait()
                gather[t + 1], offs[t + 1] = start_gather(t + 1)
            gather[t].wait()
            rv, pv = rowsb[t % 2], peb[pp]

            def row_body(r, rcarry):
                for l in range(D_MODEL // LANES):
                    sl = pl.ds(l * LANES, LANES)
                    rv[r, sl] = rv[r, sl] * SCALE + pv[r, sl]
                return rcarry

            lax.fori_loop(0, KP, row_body, 0)
            store[t] = pltpu.async_copy(rv, out_hbm.at[pl.ds(offs[t], KP)],
                                        ssb[t % 2])
        store[TASKS - 2].wait()
        store[TASKS - 1].wait()
        return carry

    lax.fori_loop(0, NP // 2, outer, 0)


def kernel(x, table):
    idx = x.reshape(TOK).astype(jnp.int32)
    out = _embed_sc(table, idx, _pe_table())
    return out.reshape(BATCH, SEQ, D_MODEL)


# speedup vs baseline: 2.4157x; 1.9173x over previous
import functools
import math

import numpy as np
import jax
import jax.numpy as jnp
from jax import lax
from jax.experimental import pallas as pl
from jax.experimental.pallas import tpu as pltpu
from jax.experimental.pallas import tpu_sc as plsc

VOCAB = 100000
D_MODEL = 768
BATCH = 4
SEQ = 8192
TOK = BATCH * SEQ
NC, NS, LANES = 2, 16, 16
NW = NC * NS
PPW = SEQ // NW
KP = 32
NP = PPW // KP
TASKS = 2 * BATCH
SCALE = math.sqrt(D_MODEL)


def _pe_table():
    pos = np.arange(SEQ, dtype=np.float32)[:, None]
    i = np.arange(D_MODEL // 2, dtype=np.float32)[None, :]
    angle = pos / np.power(10000.0, (2.0 * i) / D_MODEL, dtype=np.float32)
    pe = np.zeros((SEQ, D_MODEL), dtype=np.float32)
    pe[:, 0::2] = np.sin(angle)
    pe[:, 1::2] = np.cos(angle)
    return pe


@functools.partial(
    pl.kernel,
    mesh=plsc.VectorSubcoreMesh(core_axis_name="c", subcore_axis_name="s"),
    out_type=jax.ShapeDtypeStruct((TOK, D_MODEL), jnp.float32),
    scratch_types=[
        pltpu.VMEM((KP,), jnp.int32),
        pltpu.VMEM((KP,), jnp.int32),
        pltpu.VMEM((KP, D_MODEL), jnp.float32),
        pltpu.VMEM((KP, D_MODEL), jnp.float32),
        pltpu.VMEM((KP, D_MODEL), jnp.float32),
        pltpu.VMEM((KP, D_MODEL), jnp.float32),
        pltpu.SemaphoreType.DMA,
        pltpu.SemaphoreType.DMA,
        pltpu.SemaphoreType.DMA,
        pltpu.SemaphoreType.DMA,
        pltpu.SemaphoreType.DMA,
        pltpu.SemaphoreType.DMA,
    ],
)
def _embed_sc(table_hbm, idx_hbm, pe_hbm, out_hbm,
              idx0, idx1, rows0, rows1, pe0, pe1,
              sg0, sg1, ss0, ss1, spe0, spe1):
    idxb, rowsb, peb = [idx0, idx1], [rows0, rows1], [pe0, pe1]
    sgb, ssb, speb = [sg0, sg1], [ss0, ss1], [spe0, spe1]

    wid = lax.axis_index("s") * NC + lax.axis_index("c")
    pbase = wid * PPW

    def outer(p2, carry):
        p = 2 * p2
        pos_off = [pl.multiple_of(pbase + (p + pp) * KP, KP) for pp in range(2)]
        pe_cp = [
            pltpu.async_copy(pe_hbm.at[pl.ds(pos_off[pp], KP)], peb[pp], speb[pp])
            for pp in range(2)
        ]

        def start_gather(t):
            pp, b = t // BATCH, t % BATCH
            off = pl.multiple_of(b * SEQ + pos_off[pp], KP)
            pltpu.sync_copy(idx_hbm.at[pl.ds(off, KP)], idxb[t % 2])
            return pltpu.async_copy(table_hbm.at[idxb[t % 2]], rowsb[t % 2],
                                    sgb[t % 2]), off

        gather = [None] * TASKS
        offs = [None] * TASKS
        store = [None] * TASKS
        gather[0], offs[0] = start_gather(0)
        for t in range(TASKS):
            pp = t // BATCH
            if t % BATCH == 0:
                pe_cp[pp].wait()
            if t + 1 < TASKS:
                if t >= 1:
                    store[t - 1].wait()
                gather[t + 1], offs[t + 1] = start_gather(t + 1)
            gather[t].wait()
            rv, pv = rowsb[t % 2], peb[pp]

            def row_body(r, rcarry):
                for l in range(D_MODEL // LANES):
                    sl = pl.ds(l * LANES, LANES)
                    rv[r, sl] = rv[r, sl] * SCALE + pv[r, sl]
                return rcarry

            lax.fori_loop(0, KP, row_body, 0)
            store[t] = pltpu.async_copy(rv, out_hbm.at[pl.ds(offs[t], KP)],
                                        ssb[t % 2])
        store[TASKS - 2].wait()
        store[TASKS - 1].wait()
        return carry

    lax.fori_loop(0, NP // 2, outer, 0)


def kernel(x, table):
    idx = x.reshape(TOK).astype(jnp.int32)
    out = _embed_sc(table, idx, _pe_table())
    return out.reshape(BATCH, SEQ, D_MODEL)
